# Initial kernel scaffold; baseline (speedup 1.0000x reference)
import jax
import jax.numpy as jnp
from jax.experimental import pallas as pl


def kernel(features, edge_w, W_bond, b_bond, W1, b1, W2, b2, bn1_gamma, bn1_beta, bn2_gamma, bn2_beta, edge_index):
    raise NotImplementedError("write your pallas kernel here")


# speedup vs baseline: 3.6719x; 3.6719x over previous
import functools

import jax
import jax.numpy as jnp
from jax import lax
from jax.experimental import pallas as pl
from jax.experimental.pallas import tpu as pltpu
from jax.experimental.pallas import tpu_sc as plsc

N = 10000
E = 320000
H = 128
DE = 16
HH = H // 2

NC = 2
NS = 16
EB = 128
NBLK = E // EB
BLK_PER_S = -(-NBLK // NS)
ROWS_PER_S = 624
TAIL = N - ROWS_PER_S * NS
ZCH = 208


def _sc_segment_sums(feat_halves, src2d, dst2d, edge_w):
    mesh = plsc.VectorSubcoreMesh(core_axis_name="c", subcore_axis_name="s")

    @functools.partial(
        pl.kernel,
        mesh=mesh,
        compiler_params=pltpu.CompilerParams(use_tc_tiling_on_sc=False),
        out_type=(
            jax.ShapeDtypeStruct((NC, N, HH), jnp.float32),
            jax.ShapeDtypeStruct((N, DE), jnp.float32),
            jax.ShapeDtypeStruct((N, DE), jnp.float32),
        ),
        scratch_types=[
            pltpu.VMEM((EB,), jnp.int32),
            pltpu.VMEM((EB,), jnp.int32),
            pltpu.VMEM((EB, HH), jnp.float32),
            pltpu.VMEM((EB, DE), jnp.float32),
            pltpu.VMEM((ZCH, HH), jnp.float32),
            pltpu.VMEM((ZCH, DE), jnp.float32),
            pltpu.VMEM_SHARED((N, HH), jnp.float32),
            pltpu.VMEM_SHARED((N, DE), jnp.float32),
            pltpu.SemaphoreType.DMA,
        ],
    )
    def k(feat_hbm, src_hbm, dst_hbm, ew_hbm,
          h1_out, se_out, cnt_out,
          src_v, dst_v, rows_v, ew_v, z_w, z_n,
          acc_h1, acc_nar, sem):
        c = lax.axis_index("c")
        s = lax.axis_index("s")

        zero16 = jnp.zeros((16,), jnp.float32)
        one16 = jnp.ones((16,), jnp.float32)

        @pl.loop(0, ZCH)
        def _(i):
            @pl.loop(0, HH // 16)
            def _(l):
                z_w[i, pl.ds(l * 16, 16)] = zero16
            z_n[i, pl.ds(0, 16)] = zero16

        @pl.when(c == 1)
        def _():
            @pl.loop(0, EB)
            def _(i):
                ew_v[i, pl.ds(0, 16)] = one16

        for ch in range(ROWS_PER_S // ZCH):
            base = s * ROWS_PER_S + ch * ZCH
            pltpu.sync_copy(z_w, acc_h1.at[pl.ds(base, ZCH)])
            pltpu.sync_copy(z_n, acc_nar.at[pl.ds(base, ZCH)])

        @pl.when(s == NS - 1)
        def _():
            tb = ROWS_PER_S * NS
            pltpu.sync_copy(z_w.at[pl.ds(0, TAIL)], acc_h1.at[pl.ds(tb, TAIL)])
            pltpu.sync_copy(z_n.at[pl.ds(0, TAIL)], acc_nar.at[pl.ds(tb, TAIL)])

        plsc.subcore_barrier()

        @pl.loop(0, BLK_PER_S)
        def _(kk):
            j = kk * NS + s

            @pl.when(j < NBLK)
            def _():
                pltpu.sync_copy(src_hbm.at[j], src_v)
                pltpu.sync_copy(dst_hbm.at[j], dst_v)

                @pl.when(c == 0)
                def _():
                    pltpu.sync_copy(ew_hbm.at[pl.ds(j * EB, EB)], ew_v)

                pltpu.async_copy(feat_hbm.at[c].at[src_v], rows_v, sem).wait()
                pltpu.sync_copy(rows_v, acc_h1.at[dst_v], add=True)
                pltpu.sync_copy(ew_v, acc_nar.at[dst_v], add=True)

        plsc.subcore_barrier()

        rbase = s * ROWS_PER_S
        pltpu.sync_copy(acc_h1.at[pl.ds(rbase, ROWS_PER_S)],
                        h1_out.at[c].at[pl.ds(rbase, ROWS_PER_S)])

        @pl.when(c == 0)
        def _():
            pltpu.sync_copy(acc_nar.at[pl.ds(rbase, ROWS_PER_S)],
                            se_out.at[pl.ds(rbase, ROWS_PER_S)])

        @pl.when(c == 1)
        def _():
            pltpu.sync_copy(acc_nar.at[pl.ds(rbase, ROWS_PER_S)],
                            cnt_out.at[pl.ds(rbase, ROWS_PER_S)])

        @pl.when(s == NS - 1)
        def _():
            tb = ROWS_PER_S * NS
            pltpu.sync_copy(acc_h1.at[pl.ds(tb, TAIL)],
                            h1_out.at[c].at[pl.ds(tb, TAIL)])

            @pl.when(c == 0)
            def _():
                pltpu.sync_copy(acc_nar.at[pl.ds(tb, TAIL)],
                                se_out.at[pl.ds(tb, TAIL)])

            @pl.when(c == 1)
            def _():
                pltpu.sync_copy(acc_nar.at[pl.ds(tb, TAIL)],
                                cnt_out.at[pl.ds(tb, TAIL)])

    return k(feat_halves, src2d, dst2d, edge_w)


def _tc_epilogue(h1h, se, cnt, features, Wb_t, b_bond, W1_t, b1, W2_t, b2,
                 g1, be1, g2, be2):
    def body(h1h_ref, se_ref, cnt_ref, feat_ref, wbt_ref, bb_ref,
             w1t_ref, b1_ref, w2t_ref, b2_ref, g1_ref, be1_ref,
             g2_ref, be2_ref, y_ref):
        h1 = jnp.concatenate([h1h_ref[0], h1h_ref[1]], axis=1)
        cnt = cnt_ref[:, 0:1]
        h2 = jnp.dot(se_ref[...], wbt_ref[...],
                     preferred_element_type=jnp.float32)
        h2 = h2 + cnt * bb_ref[...]
        x = jnp.dot(h1, w1t_ref[...], preferred_element_type=jnp.float32)
        x = x + b1_ref[...] + h2
        mean1 = jnp.mean(x, axis=0, keepdims=True)
        var1 = jnp.mean(jnp.square(x), axis=0, keepdims=True) - jnp.square(mean1)
        x = (x - mean1) * lax.rsqrt(var1 + 1e-5) * g1_ref[...] + be1_ref[...]
        x = jnp.maximum(x, 0.0)
        y = jnp.dot(x, w2t_ref[...], preferred_element_type=jnp.float32)
        y = y + b2_ref[...] + feat_ref[...]
        mean2 = jnp.mean(y, axis=0, keepdims=True)
        var2 = jnp.mean(jnp.square(y), axis=0, keepdims=True) - jnp.square(mean2)
        y = (y - mean2) * lax.rsqrt(var2 + 1e-5) * g2_ref[...] + be2_ref[...]
        y_ref[...] = jnp.maximum(y, 0.0)

    return pl.pallas_call(
        body,
        out_shape=jax.ShapeDtypeStruct((N, H), jnp.float32),
    )(h1h, se, cnt, features, Wb_t, b_bond, W1_t, b1, W2_t, b2,
      g1, be1, g2, be2)


@jax.jit
def kernel(features, edge_w, W_bond, b_bond, W1, b1, W2, b2,
           bn1_gamma, bn1_beta, bn2_gamma, bn2_beta, edge_index):
    src2d = edge_index[0].astype(jnp.int32).reshape(NBLK, EB)
    dst2d = edge_index[1].astype(jnp.int32).reshape(NBLK, EB)
    feat_halves = jnp.stack([features[:, :HH], features[:, HH:]])

    h1h, se, cnt = _sc_segment_sums(feat_halves, src2d, dst2d, edge_w)

    return _tc_epilogue(
        h1h, se, cnt, features,
        W_bond.astype(jnp.float32).T, b_bond.reshape(1, H),
        W1.T, b1.reshape(1, H), W2.T, b2.reshape(1, H),
        bn1_gamma.reshape(1, H), bn1_beta.reshape(1, H),
        bn2_gamma.reshape(1, H), bn2_beta.reshape(1, H),
    )
